# baseline (speedup 1.0000x reference)
import functools

import jax
import jax.numpy as jnp
from jax import lax
from jax.experimental import pallas as pl
from jax.experimental.pallas import tpu as pltpu
from jax.experimental.pallas import tpu_sc as plsc

N = 10000
E = 320000
D = 128
NC = 2
NS = 16
NW = NC * NS
K = 64
CH = 160
EWP = CH * K
EP = NW * EWP
NP = 10240
BT = NP // NS

_mesh = plsc.VectorSubcoreMesh(core_axis_name="c", subcore_axis_name="s")


@functools.partial(
    pl.kernel,
    out_type=jax.ShapeDtypeStruct((NC, NP, D), jnp.float32),
    mesh=_mesh,
    scratch_types=[
        pltpu.VMEM((CH, K), jnp.int32),
        pltpu.VMEM((K, D), jnp.float32),
        pltpu.VMEM_SHARED((NP, D), jnp.float32),
    ],
)
def _sc_deg(dst_hbm, ones_hbm, zeros_hbm, out_hbm, idx_v, row_v, deg_sh):
    c = lax.axis_index("c")
    s = lax.axis_index("s")
    wid = s * NC + c
    pltpu.sync_copy(zeros_hbm, row_v)
    base = s * BT
    for k in range(BT // K):
        pltpu.sync_copy(row_v, deg_sh.at[pl.ds(base + k * K, K)])
    pltpu.sync_copy(dst_hbm.at[wid], idx_v)
    pltpu.sync_copy(ones_hbm, row_v)
    plsc.subcore_barrier()

    def chunk(ci, carry):
        pltpu.sync_copy(row_v, deg_sh.at[idx_v.at[ci]], add=True)
        return carry

    lax.fori_loop(0, CH, chunk, 0)
    plsc.subcore_barrier()
    pltpu.sync_copy(deg_sh.at[pl.ds(base, BT)], out_hbm.at[c, pl.ds(base, BT)])


_RING = 4


@functools.partial(
    pl.kernel,
    out_type=jax.ShapeDtypeStruct((NC, NP, D), jnp.float32),
    mesh=_mesh,
    scratch_types=(
        [pltpu.VMEM((2, K), jnp.int32) for _ in range(_RING)]
        + [pltpu.VMEM((K, D), jnp.float32) for _ in range(_RING)]
        + [pltpu.VMEM_SHARED((NP, D), jnp.float32)]
        + [pltpu.SemaphoreType.DMA] * (3 * _RING)
    ),
)
def _sc_agg(y_hbm, ei_hbm, zeros_hbm, out_hbm, *sc):
    iv = sc[0:_RING]
    bv = sc[_RING:2 * _RING]
    acc_sh = sc[2 * _RING]
    si = sc[2 * _RING + 1: 3 * _RING + 1]
    sg = sc[3 * _RING + 1: 4 * _RING + 1]
    ss = sc[4 * _RING + 1: 5 * _RING + 1]

    c = lax.axis_index("c")
    s = lax.axis_index("s")
    wid = s * NC + c
    pltpu.sync_copy(zeros_hbm, bv[0])
    base = s * BT
    for k in range(BT // K):
        pltpu.sync_copy(bv[0], acc_sh.at[pl.ds(base + k * K, K)])

    def fetch_idx(ci, k):
        pltpu.async_copy(ei_hbm.at[wid, ci], iv[k], si[k])

    def wait_idx(ci, k):
        pltpu.make_async_copy(ei_hbm.at[wid, ci], iv[k], si[k]).wait()

    def start_gather(k):
        pltpu.async_copy(y_hbm.at[iv[k].at[0]], bv[k], sg[k])

    def wait_gather(k):
        pltpu.make_async_copy(y_hbm.at[iv[k].at[0]], bv[k], sg[k]).wait()

    def start_scatter(k):
        pltpu.async_copy(bv[k], acc_sh.at[iv[k].at[1]], ss[k], add=True)

    def wait_scatter(k):
        pltpu.make_async_copy(bv[k], acc_sh.at[iv[k].at[1]], ss[k]).wait()

    fetch_idx(0, 0)
    fetch_idx(1, 1)
    plsc.subcore_barrier()
    wait_idx(0, 0)
    start_gather(0)

    def group(g, carry):
        ci0 = _RING * g
        for j in range(_RING):
            ci = ci0 + j
            wait_gather(j)
            pltpu.sync_copy(bv[j], acc_sh.at[iv[j].at[1]], add=True)
            @pl.when(ci + 1 < CH)
            def _():
                wait_idx(ci + 1, (j + 1) % _RING)
                start_gather((j + 1) % _RING)
            @pl.when(ci + 2 < CH)
            def _():
                fetch_idx(ci + 2, (j + 2) % _RING)
        return carry

    lax.fori_loop(0, CH // _RING, group, 0)
    plsc.subcore_barrier()
    pltpu.sync_copy(acc_sh.at[pl.ds(base, BT)], out_hbm.at[c, pl.ds(base, BT)])


_BLK = 1000
_GRID = N // _BLK


def _deg_r(h0, h1):
    deg = h0[:, 0:1] + h1[:, 0:1] + 1.0
    return lax.rsqrt(deg)


def _tc_a_body(h0_ref, h1_ref, x_ref, w1_ref, yp_ref):
    r = _deg_r(h0_ref[...], h1_ref[...])
    y = lax.dot_general(x_ref[...], w1_ref[...], (((1,), (1,)), ((), ())),
                        preferred_element_type=jnp.float32)
    yp_ref[...] = r * y


def _tc_b_body(p0_ref, p1_ref, yp1_ref, h0_ref, h1_ref, b1_ref, w2_ref,
               yp2_ref):
    r = _deg_r(h0_ref[...], h1_ref[...])
    s1 = r * (p0_ref[...] + p1_ref[...] + yp1_ref[...]) + b1_ref[...]
    h = jnp.maximum(s1, 0.0)
    z = lax.dot_general(h, w2_ref[...], (((1,), (1,)), ((), ())),
                        preferred_element_type=jnp.float32)
    yp2_ref[...] = r * z


def _tc_c_body(q0_ref, q1_ref, yp2_ref, h0_ref, h1_ref, b2_ref, out_ref):
    r = _deg_r(h0_ref[...], h1_ref[...])
    out_ref[...] = (r * (q0_ref[...] + q1_ref[...] + yp2_ref[...])
                    + b2_ref[...])


def _rowspec(w):
    return pl.BlockSpec((_BLK, w), lambda i: (i, 0))


def _bcast(shape):
    return pl.BlockSpec(shape, lambda i: (0,) * len(shape))


def kernel(x, edge_index, W1, b1, W2, b2):
    pad = EP - E
    fill = jnp.arange(pad, dtype=edge_index.dtype)
    src = jnp.concatenate([edge_index[0], fill % N]).reshape(NW, CH, K)
    dstf = jnp.concatenate([edge_index[1], N + fill % (NP - N)])
    dst = dstf.reshape(NW, CH, K)
    ei = jnp.stack([src, dst], axis=2)
    zerosD = jnp.zeros((K, D), jnp.float32)
    onesD = jnp.ones((K, D), jnp.float32)

    hist = _sc_deg(dst, onesD, zerosD)
    h0 = hist[0, :N]
    h1 = hist[1, :N]

    yp1 = pl.pallas_call(
        _tc_a_body,
        grid=(_GRID,),
        in_specs=[_rowspec(D), _rowspec(D), _rowspec(D), _bcast((D, D))],
        out_specs=_rowspec(D),
        out_shape=jax.ShapeDtypeStruct((N, D), jnp.float32),
    )(h0, h1, x, W1)

    P = _sc_agg(yp1, ei, zerosD)

    yp2 = pl.pallas_call(
        _tc_b_body,
        grid=(_GRID,),
        in_specs=[_rowspec(D), _rowspec(D), _rowspec(D), _rowspec(D),
                  _rowspec(D), _bcast((1, D)), _bcast((D, D))],
        out_specs=_rowspec(D),
        out_shape=jax.ShapeDtypeStruct((N, D), jnp.float32),
    )(P[0, :N], P[1, :N], yp1, h0, h1, b1.reshape(1, D), W2)

    Q = _sc_agg(yp2, ei, zerosD)

    out = pl.pallas_call(
        _tc_c_body,
        grid=(_GRID,),
        in_specs=[_rowspec(D), _rowspec(D), _rowspec(D), _rowspec(D),
                  _rowspec(D), _bcast((1, D))],
        out_specs=_rowspec(D),
        out_shape=jax.ShapeDtypeStruct((N, D), jnp.float32),
    )(Q[0, :N], Q[1, :N], yp2, h0, h1, b2.reshape(1, D))
    return out


# speedup vs baseline: 1.2139x; 1.2139x over previous
import functools

import jax
import jax.numpy as jnp
from jax import lax
from jax.experimental import pallas as pl
from jax.experimental.pallas import tpu as pltpu
from jax.experimental.pallas import tpu_sc as plsc

N = 10000
E = 320000
D = 128
NC = 2
NS = 16
NW = NC * NS
K = 64
CH = 160
EWP = CH * K
EP = NW * EWP
NP = 10240
BT = NP // NS

_mesh = plsc.VectorSubcoreMesh(core_axis_name="c", subcore_axis_name="s")


@functools.partial(
    pl.kernel,
    out_type=jax.ShapeDtypeStruct((NC, NP, D), jnp.float32),
    mesh=_mesh,
    scratch_types=[
        pltpu.VMEM((CH, K), jnp.int32),
        pltpu.VMEM((K, D), jnp.float32),
        pltpu.VMEM_SHARED((NP, D), jnp.float32),
        pltpu.SemaphoreType.DMA,
        pltpu.SemaphoreType.DMA,
    ],
)
def _sc_deg(dst_hbm, ones_hbm, zeros_hbm, out_hbm, idx_v, row_v, deg_sh,
            ds0, ds1):
    c = lax.axis_index("c")
    s = lax.axis_index("s")
    wid = s * NC + c
    pltpu.sync_copy(zeros_hbm, row_v)
    base = s * BT
    for k in range(BT // K):
        pltpu.sync_copy(row_v, deg_sh.at[pl.ds(base + k * K, K)])
    pltpu.sync_copy(dst_hbm.at[wid], idx_v)
    pltpu.sync_copy(ones_hbm, row_v)
    plsc.subcore_barrier()

    def sca(ci, sem):
        pltpu.async_copy(row_v, deg_sh.at[idx_v.at[ci]], sem, add=True)

    def wsca(ci, sem):
        pltpu.make_async_copy(row_v, deg_sh.at[idx_v.at[ci]], sem).wait()

    def pair(i, carry):
        ci = 2 * i
        @pl.when(i > 0)
        def _():
            wsca(ci - 2, ds0)
        sca(ci, ds0)
        @pl.when(i > 0)
        def _():
            wsca(ci - 1, ds1)
        sca(ci + 1, ds1)
        return carry

    lax.fori_loop(0, CH // 2, pair, 0)
    wsca(CH - 2, ds0)
    wsca(CH - 1, ds1)
    plsc.subcore_barrier()
    pltpu.sync_copy(deg_sh.at[pl.ds(base, BT)], out_hbm.at[c, pl.ds(base, BT)])


_RING = 4


@functools.partial(
    pl.kernel,
    out_type=jax.ShapeDtypeStruct((NC, NP, D), jnp.float32),
    mesh=_mesh,
    scratch_types=(
        [pltpu.VMEM((2, K), jnp.int32) for _ in range(_RING)]
        + [pltpu.VMEM((K, D), jnp.float32) for _ in range(_RING)]
        + [pltpu.VMEM_SHARED((NP, D), jnp.float32)]
        + [pltpu.SemaphoreType.DMA] * (3 * _RING)
    ),
)
def _sc_agg(y_hbm, ei_hbm, zeros_hbm, out_hbm, *sc):
    iv = sc[0:_RING]
    bv = sc[_RING:2 * _RING]
    acc_sh = sc[2 * _RING]
    si = sc[2 * _RING + 1: 3 * _RING + 1]
    sg = sc[3 * _RING + 1: 4 * _RING + 1]
    ss = sc[4 * _RING + 1: 5 * _RING + 1]

    c = lax.axis_index("c")
    s = lax.axis_index("s")
    wid = s * NC + c
    pltpu.sync_copy(zeros_hbm, bv[0])
    base = s * BT
    for k in range(BT // K):
        pltpu.sync_copy(bv[0], acc_sh.at[pl.ds(base + k * K, K)])

    def fetch_idx(ci, k):
        pltpu.async_copy(ei_hbm.at[wid, ci], iv[k], si[k])

    def wait_idx(ci, k):
        pltpu.make_async_copy(ei_hbm.at[wid, ci], iv[k], si[k]).wait()

    def start_gather(k):
        pltpu.async_copy(y_hbm.at[iv[k].at[0]], bv[k], sg[k])

    def wait_gather(k):
        pltpu.make_async_copy(y_hbm.at[iv[k].at[0]], bv[k], sg[k]).wait()

    def start_scatter(k):
        pltpu.async_copy(bv[k], acc_sh.at[iv[k].at[1]], ss[k], add=True)

    def wait_scatter(k):
        pltpu.make_async_copy(bv[k], acc_sh.at[iv[k].at[1]], ss[k]).wait()

    fetch_idx(0, 0)
    fetch_idx(1, 1)
    plsc.subcore_barrier()
    wait_idx(0, 0)
    start_gather(0)

    def group(g, carry):
        ci0 = _RING * g
        for j in range(_RING):
            ci = ci0 + j
            wait_gather(j)
            start_scatter(j)
            @pl.when(ci + 1 < CH)
            def _():
                wait_idx(ci + 1, (j + 1) % _RING)
                start_gather((j + 1) % _RING)
            @pl.when(ci + 2 < CH)
            def _():
                @pl.when(ci >= 2)
                def _():
                    wait_scatter((j + 2) % _RING)
                fetch_idx(ci + 2, (j + 2) % _RING)
        return carry

    lax.fori_loop(0, CH // _RING, group, 0)
    wait_scatter((CH - 2) % _RING)
    wait_scatter((CH - 1) % _RING)
    plsc.subcore_barrier()
    pltpu.sync_copy(acc_sh.at[pl.ds(base, BT)], out_hbm.at[c, pl.ds(base, BT)])


_BLK = 1000
_GRID = N // _BLK


def _deg_r(h0, h1):
    deg = h0[:, 0:1] + h1[:, 0:1] + 1.0
    return lax.rsqrt(deg)


def _tc_a_body(h0_ref, h1_ref, x_ref, w1_ref, yp_ref):
    r = _deg_r(h0_ref[...], h1_ref[...])
    y = lax.dot_general(x_ref[...], w1_ref[...], (((1,), (1,)), ((), ())),
                        preferred_element_type=jnp.float32)
    yp_ref[...] = r * y


def _tc_b_body(p0_ref, p1_ref, yp1_ref, h0_ref, h1_ref, b1_ref, w2_ref,
               yp2_ref):
    r = _deg_r(h0_ref[...], h1_ref[...])
    s1 = r * (p0_ref[...] + p1_ref[...] + yp1_ref[...]) + b1_ref[...]
    h = jnp.maximum(s1, 0.0)
    z = lax.dot_general(h, w2_ref[...], (((1,), (1,)), ((), ())),
                        preferred_element_type=jnp.float32)
    yp2_ref[...] = r * z


def _tc_c_body(q0_ref, q1_ref, yp2_ref, h0_ref, h1_ref, b2_ref, out_ref):
    r = _deg_r(h0_ref[...], h1_ref[...])
    out_ref[...] = (r * (q0_ref[...] + q1_ref[...] + yp2_ref[...])
                    + b2_ref[...])


def _rowspec(w):
    return pl.BlockSpec((_BLK, w), lambda i: (i, 0))


def _bcast(shape):
    return pl.BlockSpec(shape, lambda i: (0,) * len(shape))


def kernel(x, edge_index, W1, b1, W2, b2):
    pad = EP - E
    fill = jnp.arange(pad, dtype=edge_index.dtype)
    src = jnp.concatenate([edge_index[0], fill % N]).reshape(NW, CH, K)
    dstf = jnp.concatenate([edge_index[1], N + fill % (NP - N)])
    dst = dstf.reshape(NW, CH, K)
    ei = jnp.stack([src, dst], axis=2)
    zerosD = jnp.zeros((K, D), jnp.float32)
    onesD = jnp.ones((K, D), jnp.float32)

    hist = _sc_deg(dst, onesD, zerosD)
    h0 = hist[0, :N]
    h1 = hist[1, :N]

    yp1 = pl.pallas_call(
        _tc_a_body,
        grid=(_GRID,),
        in_specs=[_rowspec(D), _rowspec(D), _rowspec(D), _bcast((D, D))],
        out_specs=_rowspec(D),
        out_shape=jax.ShapeDtypeStruct((N, D), jnp.float32),
    )(h0, h1, x, W1)

    P = _sc_agg(yp1, ei, zerosD)

    yp2 = pl.pallas_call(
        _tc_b_body,
        grid=(_GRID,),
        in_specs=[_rowspec(D), _rowspec(D), _rowspec(D), _rowspec(D),
                  _rowspec(D), _bcast((1, D)), _bcast((D, D))],
        out_specs=_rowspec(D),
        out_shape=jax.ShapeDtypeStruct((N, D), jnp.float32),
    )(P[0, :N], P[1, :N], yp1, h0, h1, b1.reshape(1, D), W2)

    Q = _sc_agg(yp2, ei, zerosD)

    out = pl.pallas_call(
        _tc_c_body,
        grid=(_GRID,),
        in_specs=[_rowspec(D), _rowspec(D), _rowspec(D), _rowspec(D),
                  _rowspec(D), _bcast((1, D))],
        out_specs=_rowspec(D),
        out_shape=jax.ShapeDtypeStruct((N, D), jnp.float32),
    )(Q[0, :N], Q[1, :N], yp2, h0, h1, b2.reshape(1, D))
    return out


# speedup vs baseline: 1.3987x; 1.1522x over previous
import functools

import jax
import jax.numpy as jnp
from jax import lax
from jax.experimental import pallas as pl
from jax.experimental.pallas import tpu as pltpu
from jax.experimental.pallas import tpu_sc as plsc

N = 10000
E = 320000
D = 128
NC = 2
NS = 16
NW = NC * NS
K = 80
CH = 128
DK = 128
DCH = 80
EWP = CH * K
EP = NW * EWP
NP = 10240
BT = NP // NS

_mesh = plsc.VectorSubcoreMesh(core_axis_name="c", subcore_axis_name="s")


@functools.partial(
    pl.kernel,
    out_type=jax.ShapeDtypeStruct((NC, NP, D), jnp.float32),
    mesh=_mesh,
    scratch_types=[
        pltpu.VMEM((DCH, DK), jnp.int32),
        pltpu.VMEM((DK, D), jnp.float32),
        pltpu.VMEM_SHARED((NP, D), jnp.float32),
        pltpu.SemaphoreType.DMA,
        pltpu.SemaphoreType.DMA,
    ],
)
def _sc_deg(dst_hbm, ones_hbm, zeros_hbm, out_hbm, idx_v, row_v, deg_sh,
            ds0, ds1):
    c = lax.axis_index("c")
    s = lax.axis_index("s")
    wid = s * NC + c
    pltpu.sync_copy(zeros_hbm, row_v)
    base = s * BT
    for k in range(BT // DK):
        pltpu.sync_copy(row_v, deg_sh.at[pl.ds(base + k * DK, DK)])
    pltpu.sync_copy(dst_hbm.at[wid], idx_v)
    pltpu.sync_copy(ones_hbm, row_v)
    plsc.subcore_barrier()

    def sca(ci, sem):
        pltpu.async_copy(row_v, deg_sh.at[idx_v.at[ci]], sem, add=True)

    def wsca(ci, sem):
        pltpu.make_async_copy(row_v, deg_sh.at[idx_v.at[ci]], sem).wait()

    def pair(i, carry):
        ci = 2 * i
        @pl.when(i > 0)
        def _():
            wsca(ci - 2, ds0)
        sca(ci, ds0)
        @pl.when(i > 0)
        def _():
            wsca(ci - 1, ds1)
        sca(ci + 1, ds1)
        return carry

    lax.fori_loop(0, DCH // 2, pair, 0)
    wsca(DCH - 2, ds0)
    wsca(DCH - 1, ds1)
    plsc.subcore_barrier()
    pltpu.sync_copy(deg_sh.at[pl.ds(base, BT)], out_hbm.at[c, pl.ds(base, BT)])


_RING = 4


@functools.partial(
    pl.kernel,
    out_type=jax.ShapeDtypeStruct((NC, NP, D), jnp.float32),
    mesh=_mesh,
    scratch_types=(
        [pltpu.VMEM((2, K), jnp.int32) for _ in range(_RING)]
        + [pltpu.VMEM((K, D), jnp.float32) for _ in range(_RING)]
        + [pltpu.VMEM_SHARED((NP, D), jnp.float32)]
        + [pltpu.SemaphoreType.DMA] * (3 * _RING)
    ),
)
def _sc_agg(y_hbm, ei_hbm, zeros_hbm, out_hbm, *sc):
    iv = sc[0:_RING]
    bv = sc[_RING:2 * _RING]
    acc_sh = sc[2 * _RING]
    si = sc[2 * _RING + 1: 3 * _RING + 1]
    sg = sc[3 * _RING + 1: 4 * _RING + 1]
    ss = sc[4 * _RING + 1: 5 * _RING + 1]

    c = lax.axis_index("c")
    s = lax.axis_index("s")
    wid = s * NC + c
    pltpu.sync_copy(zeros_hbm, bv[0])
    base = s * BT
    for k in range(BT // K):
        pltpu.sync_copy(bv[0], acc_sh.at[pl.ds(base + k * K, K)])

    def fetch_idx(ci, k):
        pltpu.async_copy(ei_hbm.at[wid, ci], iv[k], si[k])

    def wait_idx(ci, k):
        pltpu.make_async_copy(ei_hbm.at[wid, ci], iv[k], si[k]).wait()

    def start_gather(k):
        pltpu.async_copy(y_hbm.at[iv[k].at[0]], bv[k], sg[k])

    def wait_gather(k):
        pltpu.make_async_copy(y_hbm.at[iv[k].at[0]], bv[k], sg[k]).wait()

    def start_scatter(k):
        pltpu.async_copy(bv[k], acc_sh.at[iv[k].at[1]], ss[k], add=True)

    def wait_scatter(k):
        pltpu.make_async_copy(bv[k], acc_sh.at[iv[k].at[1]], ss[k]).wait()

    fetch_idx(0, 0)
    fetch_idx(1, 1)
    plsc.subcore_barrier()
    wait_idx(0, 0)
    start_gather(0)

    def group(g, carry):
        ci0 = _RING * g
        for j in range(_RING):
            ci = ci0 + j
            wait_gather(j)
            start_scatter(j)
            @pl.when(ci + 1 < CH)
            def _():
                wait_idx(ci + 1, (j + 1) % _RING)
                start_gather((j + 1) % _RING)
            @pl.when(ci + 2 < CH)
            def _():
                @pl.when(ci >= 2)
                def _():
                    wait_scatter((j + 2) % _RING)
                fetch_idx(ci + 2, (j + 2) % _RING)
        return carry

    lax.fori_loop(0, CH // _RING, group, 0)
    wait_scatter((CH - 2) % _RING)
    wait_scatter((CH - 1) % _RING)
    plsc.subcore_barrier()
    pltpu.sync_copy(acc_sh.at[pl.ds(base, BT)], out_hbm.at[c, pl.ds(base, BT)])


_BLK = 1000
_GRID = N // _BLK


def _deg_r(h0, h1):
    deg = h0[:, 0:1] + h1[:, 0:1] + 1.0
    return lax.rsqrt(deg)


def _tc_a_body(h0_ref, h1_ref, x_ref, w1_ref, yp_ref):
    r = _deg_r(h0_ref[...], h1_ref[...])
    y = lax.dot_general(x_ref[...], w1_ref[...], (((1,), (1,)), ((), ())),
                        preferred_element_type=jnp.float32)
    yp_ref[...] = r * y


def _tc_b_body(p0_ref, p1_ref, yp1_ref, h0_ref, h1_ref, b1_ref, w2_ref,
               yp2_ref):
    r = _deg_r(h0_ref[...], h1_ref[...])
    s1 = r * (p0_ref[...] + p1_ref[...] + yp1_ref[...]) + b1_ref[...]
    h = jnp.maximum(s1, 0.0)
    z = lax.dot_general(h, w2_ref[...], (((1,), (1,)), ((), ())),
                        preferred_element_type=jnp.float32)
    yp2_ref[...] = r * z


def _tc_c_body(q0_ref, q1_ref, yp2_ref, h0_ref, h1_ref, b2_ref, out_ref):
    r = _deg_r(h0_ref[...], h1_ref[...])
    out_ref[...] = (r * (q0_ref[...] + q1_ref[...] + yp2_ref[...])
                    + b2_ref[...])


def _rowspec(w):
    return pl.BlockSpec((_BLK, w), lambda i: (i, 0))


def _bcast(shape):
    return pl.BlockSpec(shape, lambda i: (0,) * len(shape))


def kernel(x, edge_index, W1, b1, W2, b2):
    pad = EP - E
    fill = jnp.arange(pad, dtype=edge_index.dtype)
    src = jnp.concatenate([edge_index[0], fill % N]).reshape(NW, CH, K)
    dstf = jnp.concatenate([edge_index[1], N + fill % (NP - N)])
    dst = dstf.reshape(NW, CH, K)
    ei = jnp.stack([src, dst], axis=2)
    zerosD = jnp.zeros((K, D), jnp.float32)
    onesDK = jnp.ones((DK, D), jnp.float32)
    zerosDK = jnp.zeros((DK, D), jnp.float32)

    hist = _sc_deg(dstf.reshape(NW, DCH, DK), onesDK, zerosDK)
    h0 = hist[0, :N]
    h1 = hist[1, :N]

    yp1 = pl.pallas_call(
        _tc_a_body,
        grid=(_GRID,),
        in_specs=[_rowspec(D), _rowspec(D), _rowspec(D), _bcast((D, D))],
        out_specs=_rowspec(D),
        out_shape=jax.ShapeDtypeStruct((N, D), jnp.float32),
    )(h0, h1, x, W1)

    P = _sc_agg(yp1, ei, zerosD)

    yp2 = pl.pallas_call(
        _tc_b_body,
        grid=(_GRID,),
        in_specs=[_rowspec(D), _rowspec(D), _rowspec(D), _rowspec(D),
                  _rowspec(D), _bcast((1, D)), _bcast((D, D))],
        out_specs=_rowspec(D),
        out_shape=jax.ShapeDtypeStruct((N, D), jnp.float32),
    )(P[0, :N], P[1, :N], yp1, h0, h1, b1.reshape(1, D), W2)

    Q = _sc_agg(yp2, ei, zerosD)

    out = pl.pallas_call(
        _tc_c_body,
        grid=(_GRID,),
        in_specs=[_rowspec(D), _rowspec(D), _rowspec(D), _rowspec(D),
                  _rowspec(D), _bcast((1, D))],
        out_specs=_rowspec(D),
        out_shape=jax.ShapeDtypeStruct((N, D), jnp.float32),
    )(Q[0, :N], Q[1, :N], yp2, h0, h1, b2.reshape(1, D))
    return out
